# baseline (speedup 1.0000x reference)
import functools

import jax
import jax.numpy as jnp
from jax import lax
from jax.experimental import pallas as pl
from jax.experimental.pallas import tpu as pltpu
from jax.experimental.pallas import tpu_sc as plsc

N = 10000
E = 320000
D = 128
C = 40
CP = 48
W = 128

NC = 2
NS = 16
NW = NC * NS
EPW = E // NW
BC = 624
NBC = EPW // BC
TAIL = EPW - NBC * BC


def _proj_body(x_ref, wlt_ref, y_ref):
    y = lax.dot_general(x_ref[...], wlt_ref[...], (((1,), (1,)), ((), ())),
                        preferred_element_type=jnp.float32)
    y48 = jnp.pad(y, ((0, 0), (0, CP - C)))
    col = lax.broadcasted_iota(jnp.int32, y48.shape, 1)
    y_ref[...] = y48 + jnp.where(col == C, 1.0, 0.0)


def _project(x, wlt):
    blk = 2000
    return pl.pallas_call(
        _proj_body,
        grid=(N // blk,),
        in_specs=[
            pl.BlockSpec((blk, D), lambda i: (i, 0)),
            pl.BlockSpec((C, D), lambda i: (0, 0)),
        ],
        out_specs=pl.BlockSpec((blk, CP), lambda i: (i, 0)),
        out_shape=jax.ShapeDtypeStruct((N, CP), jnp.float32),
    )(x, wlt)


def _sc_aggregate(y, ei_flat, zeros):
    mesh = plsc.VectorSubcoreMesh(core_axis_name="c", subcore_axis_name="s")

    @functools.partial(
        pl.kernel,
        mesh=mesh,
        compiler_params=pltpu.CompilerParams(use_tc_tiling_on_sc=False),
        out_type=[
            jax.ShapeDtypeStruct((N, W), jnp.float32),
            jax.ShapeDtypeStruct((N, W), jnp.float32),
        ],
        scratch_types=[
            pltpu.VMEM((EPW,), jnp.int32),
            pltpu.VMEM((EPW,), jnp.int32),
            pltpu.VMEM((BC, CP), jnp.float32),
            pltpu.VMEM((BC, CP), jnp.float32),
            pltpu.VMEM_SHARED((N, CP), jnp.float32),
            pltpu.SemaphoreType.DMA,
            pltpu.SemaphoreType.DMA,
        ],
    )
    def k(y_hbm, ei_hbm, z_hbm, out0, out1,
          src_v, dst_v, buf0, buf1, agg, sem0, sem1):
        cid = lax.axis_index("c")
        sid = lax.axis_index("s")
        wid = cid * NS + sid
        st_lo = pl.ds(pl.multiple_of(sid * 624, 8), 624)
        st_hi = pl.ds(15 * 624, N - 15 * 624)

        @pl.when(sid < 15)
        def _():
            pltpu.sync_copy(z_hbm.at[pl.ds(0, 624)], agg.at[st_lo])

        @pl.when(sid == 15)
        def _():
            pltpu.sync_copy(z_hbm, agg.at[st_hi])

        pltpu.sync_copy(ei_hbm.at[pl.ds(wid * EPW, EPW)], src_v)
        pltpu.sync_copy(ei_hbm.at[pl.ds(E + wid * EPW, EPW)], dst_v)
        plsc.subcore_barrier()

        pltpu.async_copy(y_hbm.at[src_v.at[pl.ds(0, BC)]], buf0, sem0)

        def body(t, carry):
            r0 = 2 * BC * t
            pltpu.async_copy(y_hbm.at[src_v.at[pl.ds(r0 + BC, BC)]], buf1, sem1)
            pltpu.make_async_copy(
                y_hbm.at[src_v.at[pl.ds(r0, BC)]], buf0, sem0).wait()
            pltpu.sync_copy(buf0, agg.at[dst_v.at[pl.ds(r0, BC)]], add=True)

            @pl.when(r0 + 2 * BC < NBC * BC)
            def _():
                pltpu.async_copy(
                    y_hbm.at[src_v.at[pl.ds(r0 + 2 * BC, BC)]], buf0, sem0)

            pltpu.make_async_copy(
                y_hbm.at[src_v.at[pl.ds(r0 + BC, BC)]], buf1, sem1).wait()
            pltpu.sync_copy(buf1, agg.at[dst_v.at[pl.ds(r0 + BC, BC)]], add=True)
            return carry

        lax.fori_loop(0, NBC // 2, body, 0)

        bslice = buf0.at[pl.ds(0, TAIL)]
        pltpu.async_copy(
            y_hbm.at[src_v.at[pl.ds(NBC * BC, TAIL)]], bslice, sem0).wait()
        pltpu.sync_copy(bslice, agg.at[dst_v.at[pl.ds(NBC * BC, TAIL)]], add=True)

        plsc.subcore_barrier()

        @pl.when(jnp.logical_and(cid == 0, sid < 15))
        def _():
            pltpu.sync_copy(agg.at[st_lo], out0.at[st_lo, pl.ds(0, CP)])

        @pl.when(jnp.logical_and(cid == 0, sid == 15))
        def _():
            pltpu.sync_copy(agg.at[st_hi], out0.at[st_hi, pl.ds(0, CP)])

        @pl.when(jnp.logical_and(cid == 1, sid < 15))
        def _():
            pltpu.sync_copy(agg.at[st_lo], out1.at[st_lo, pl.ds(0, CP)])

        @pl.when(jnp.logical_and(cid == 1, sid == 15))
        def _():
            pltpu.sync_copy(agg.at[st_hi], out1.at[st_hi, pl.ds(0, CP)])

    return k(y, ei_flat, zeros)


def _combine_body(p0_ref, p1_ref, x_ref, wr_ref, b_ref, o_ref):
    agg = p0_ref[...] + p1_ref[...]
    col = lax.broadcasted_iota(jnp.int32, agg.shape, 1)
    cnt = jnp.sum(jnp.where(col == C, agg, 0.0), axis=1, keepdims=True)
    mean = agg / jnp.maximum(cnt, 1.0)
    z = lax.dot_general(x_ref[...], wr_ref[...], (((1,), (1,)), ((), ())),
                        preferred_element_type=jnp.float32)
    out = jnp.maximum(mean[:, :C] + z + b_ref[...], 0.0)
    o_ref[...] = out.T


def _combine(p0, p1, x, wr, b40):
    blk = 2048
    return pl.pallas_call(
        _combine_body,
        grid=(pl.cdiv(N, blk),),
        in_specs=[
            pl.BlockSpec((blk, W), lambda i: (i, 0)),
            pl.BlockSpec((blk, W), lambda i: (i, 0)),
            pl.BlockSpec((blk, D), lambda i: (i, 0)),
            pl.BlockSpec((C, D), lambda i: (0, 0)),
            pl.BlockSpec((1, C), lambda i: (0, 0)),
        ],
        out_specs=pl.BlockSpec((C, blk), lambda i: (0, i)),
        out_shape=jax.ShapeDtypeStruct((C, N), jnp.float32),
    )(p0, p1, x, wr, b40)


def kernel(x, edge_index, W_l, W_r, b):
    ei_flat = edge_index.astype(jnp.int32).reshape(2 * E)
    b40 = b.reshape(1, C)
    zeros = jnp.zeros((640, CP), jnp.float32)
    y = _project(x, W_l.T)
    p0, p1 = _sc_aggregate(y, ei_flat, zeros)
    return _combine(p0, p1, x, W_r.T, b40).T


# speedup vs baseline: 1.0348x; 1.0348x over previous
import functools

import jax
import jax.numpy as jnp
from jax import lax
from jax.experimental import pallas as pl
from jax.experimental.pallas import tpu as pltpu
from jax.experimental.pallas import tpu_sc as plsc

N = 10000
E = 320000
D = 128
C = 40
CP = 48
W = 128

NC = 2
NS = 16
NW = NC * NS
EPW = E // NW
BC = 624
NBC = EPW // BC
TAIL = EPW - NBC * BC


def _proj_body(x_ref, wl_ref, y_ref):
    y = jnp.dot(x_ref[...], wl_ref[...], preferred_element_type=jnp.float32)
    col = lax.broadcasted_iota(jnp.int32, y.shape, 1)
    y = y + jnp.where(col == C, 1.0, 0.0)
    y3 = y.reshape(y.shape[0] // 8, 8, CP)

    def piece(i, lo, hi, shift):
        seg = y3[:, i, lo:hi]
        return jnp.pad(seg, ((0, 0), (shift, 128 - shift - (hi - lo))))

    q0 = piece(0, 0, 48, 0) + piece(1, 0, 48, 48) + piece(2, 0, 32, 96)
    q1 = (piece(2, 32, 48, 0) + piece(3, 0, 48, 16) + piece(4, 0, 48, 64)
          + piece(5, 0, 16, 112))
    q2 = piece(5, 16, 48, 0) + piece(6, 0, 48, 32) + piece(7, 0, 48, 80)
    out = jnp.stack([q0, q1, q2], axis=1)
    y_ref[...] = out.reshape(out.shape[0] * 3, 128)


def _project(x, wl48):
    blk = 2048
    oblk = blk * CP // 128
    return pl.pallas_call(
        _proj_body,
        grid=(pl.cdiv(N, blk),),
        in_specs=[
            pl.BlockSpec((blk, D), lambda i: (i, 0)),
            pl.BlockSpec((D, CP), lambda i: (0, 0)),
        ],
        out_specs=pl.BlockSpec((oblk, 128), lambda i: (i, 0)),
        out_shape=jax.ShapeDtypeStruct((N * CP // 128, 128), jnp.float32),
    )(x, wl48)


def _sc_aggregate(y, ei_flat, zeros):
    mesh = plsc.VectorSubcoreMesh(core_axis_name="c", subcore_axis_name="s")

    @functools.partial(
        pl.kernel,
        mesh=mesh,
        compiler_params=pltpu.CompilerParams(use_tc_tiling_on_sc=False),
        out_type=[
            jax.ShapeDtypeStruct((N, W), jnp.float32),
            jax.ShapeDtypeStruct((N, W), jnp.float32),
        ],
        scratch_types=[
            pltpu.VMEM((EPW,), jnp.int32),
            pltpu.VMEM((EPW,), jnp.int32),
            pltpu.VMEM((BC, CP), jnp.float32),
            pltpu.VMEM((BC, CP), jnp.float32),
            pltpu.VMEM_SHARED((N, CP), jnp.float32),
            pltpu.SemaphoreType.DMA,
            pltpu.SemaphoreType.DMA,
        ],
    )
    def k(y_hbm, ei_hbm, z_hbm, out0, out1,
          src_v, dst_v, buf0, buf1, agg, sem0, sem1):
        cid = lax.axis_index("c")
        sid = lax.axis_index("s")
        wid = cid * NS + sid
        st_lo = pl.ds(pl.multiple_of(sid * 624, 8), 624)
        st_hi = pl.ds(15 * 624, N - 15 * 624)

        @pl.when(sid < 15)
        def _():
            pltpu.sync_copy(z_hbm.at[pl.ds(0, 624)], agg.at[st_lo])

        @pl.when(sid == 15)
        def _():
            pltpu.sync_copy(z_hbm, agg.at[st_hi])

        pltpu.sync_copy(ei_hbm.at[pl.ds(wid * EPW, EPW)], src_v)
        pltpu.sync_copy(ei_hbm.at[pl.ds(E + wid * EPW, EPW)], dst_v)
        plsc.subcore_barrier()

        pltpu.async_copy(y_hbm.at[src_v.at[pl.ds(0, BC)]], buf0, sem0)

        def body(t, carry):
            r0 = 2 * BC * t
            pltpu.async_copy(y_hbm.at[src_v.at[pl.ds(r0 + BC, BC)]], buf1, sem1)
            pltpu.make_async_copy(
                y_hbm.at[src_v.at[pl.ds(r0, BC)]], buf0, sem0).wait()
            pltpu.sync_copy(buf0, agg.at[dst_v.at[pl.ds(r0, BC)]], add=True)

            @pl.when(r0 + 2 * BC < NBC * BC)
            def _():
                pltpu.async_copy(
                    y_hbm.at[src_v.at[pl.ds(r0 + 2 * BC, BC)]], buf0, sem0)

            pltpu.make_async_copy(
                y_hbm.at[src_v.at[pl.ds(r0 + BC, BC)]], buf1, sem1).wait()
            pltpu.sync_copy(buf1, agg.at[dst_v.at[pl.ds(r0 + BC, BC)]], add=True)
            return carry

        lax.fori_loop(0, NBC // 2, body, 0)

        bslice = buf0.at[pl.ds(0, TAIL)]
        pltpu.async_copy(
            y_hbm.at[src_v.at[pl.ds(NBC * BC, TAIL)]], bslice, sem0).wait()
        pltpu.sync_copy(bslice, agg.at[dst_v.at[pl.ds(NBC * BC, TAIL)]], add=True)

        plsc.subcore_barrier()

        @pl.when(jnp.logical_and(cid == 0, sid < 15))
        def _():
            pltpu.sync_copy(agg.at[st_lo], out0.at[st_lo, pl.ds(0, CP)])

        @pl.when(jnp.logical_and(cid == 0, sid == 15))
        def _():
            pltpu.sync_copy(agg.at[st_hi], out0.at[st_hi, pl.ds(0, CP)])

        @pl.when(jnp.logical_and(cid == 1, sid < 15))
        def _():
            pltpu.sync_copy(agg.at[st_lo], out1.at[st_lo, pl.ds(0, CP)])

        @pl.when(jnp.logical_and(cid == 1, sid == 15))
        def _():
            pltpu.sync_copy(agg.at[st_hi], out1.at[st_hi, pl.ds(0, CP)])

    return k(y, ei_flat, zeros)


def _combine_body(p0_ref, p1_ref, x_ref, wr_ref, b_ref, o_ref):
    agg = p0_ref[...] + p1_ref[...]
    col = lax.broadcasted_iota(jnp.int32, agg.shape, 1)
    cnt = jnp.sum(jnp.where(col == C, agg, 0.0), axis=1, keepdims=True)
    mean = agg / jnp.maximum(cnt, 1.0)
    z = jnp.dot(x_ref[...], wr_ref[...], preferred_element_type=jnp.float32)
    out = jnp.maximum(mean[:, :C] + z + b_ref[...], 0.0)
    o_ref[...] = out.T


def _combine(p0, p1, x, wr, b40):
    blk = 2048
    return pl.pallas_call(
        _combine_body,
        grid=(pl.cdiv(N, blk),),
        in_specs=[
            pl.BlockSpec((blk, W), lambda i: (i, 0)),
            pl.BlockSpec((blk, W), lambda i: (i, 0)),
            pl.BlockSpec((blk, D), lambda i: (i, 0)),
            pl.BlockSpec((D, C), lambda i: (0, 0)),
            pl.BlockSpec((1, C), lambda i: (0, 0)),
        ],
        out_specs=pl.BlockSpec((C, blk), lambda i: (0, i)),
        out_shape=jax.ShapeDtypeStruct((C, N), jnp.float32),
    )(p0, p1, x, wr, b40)


def kernel(x, edge_index, W_l, W_r, b):
    ei_flat = edge_index.astype(jnp.int32).reshape(2 * E)
    wl48 = jnp.pad(W_l, ((0, 0), (0, CP - C)))
    b40 = b.reshape(1, C)
    zeros = jnp.zeros((640, CP), jnp.float32)
    y = _project(x, wl48).reshape(N, CP)
    p0, p1 = _sc_aggregate(y, ei_flat, zeros)
    return _combine(p0, p1, x, W_r, b40).T
